# baseline (speedup 1.0000x reference)
import functools

import jax
import jax.numpy as jnp
from jax import lax
from jax.experimental import pallas as pl
from jax.experimental.pallas import tpu as pltpu
from jax.experimental.pallas import tpu_sc as plsc

N = 10000
E = 320000
D = 128
F = 16
LANES = 16
NC, NS = 2, 16
NW = NC * NS
EPW = E // NW
CH = 80
NCH = EPW // CH
NPAD = 10240
NPT = NPAD // NS

_mesh = lambda: plsc.VectorSubcoreMesh(core_axis_name="c", subcore_axis_name="s")


def _pq_body(x_ref, wr_ref, wc_ref, p_ref, q_ref):
    xv = x_ref[...].astype(jnp.bfloat16)
    p_ref[...] = jnp.dot(xv, wr_ref[...].astype(jnp.bfloat16),
                         preferred_element_type=jnp.float32)
    q_ref[...] = jnp.dot(xv, wc_ref[...].astype(jnp.bfloat16),
                         preferred_element_type=jnp.float32)


def _pq(x, wr, wc):
    return pl.pallas_call(
        _pq_body,
        out_shape=[jax.ShapeDtypeStruct((N, D), jnp.float32),
                   jax.ShapeDtypeStruct((N, D), jnp.float32)],
    )(x, wr, wc)


def _gather_body(p_hbm, q_hbm, row_hbm, col_hbm, out_hbm,
                 idxr, idxc, buf_p, buf_q, sem_p, sem_q):
    wid = lax.axis_index("s") * NC + lax.axis_index("c")
    base = wid * EPW

    def chunk(k, carry):
        off = base + k * CH
        pltpu.sync_copy(row_hbm.at[pl.ds(off, CH)], idxr)
        pltpu.sync_copy(col_hbm.at[pl.ds(off, CH)], idxc)
        cp_p = pltpu.async_copy(p_hbm.at[idxr], buf_p, sem_p)
        cp_q = pltpu.async_copy(q_hbm.at[idxc], buf_q, sem_q)
        cp_p.wait()
        cp_q.wait()

        def addrow(e, c2):
            for j in range(D // LANES):
                sl = pl.ds(j * LANES, LANES)
                buf_p[e, sl] = buf_p[e, sl] + buf_q[e, sl]
            return c2

        lax.fori_loop(0, CH, addrow, None)
        pltpu.sync_copy(buf_p, out_hbm.at[pl.ds(off, CH)])
        return carry

    lax.fori_loop(0, NCH, chunk, None)


def _gather(p, q, row, col):
    fn = pl.kernel(
        _gather_body,
        out_type=jax.ShapeDtypeStruct((E, D), jnp.float32),
        mesh=_mesh(),
        scratch_types=[
            pltpu.VMEM((CH,), jnp.int32),
            pltpu.VMEM((CH,), jnp.int32),
            pltpu.VMEM((CH, D), jnp.float32),
            pltpu.VMEM((CH, D), jnp.float32),
            pltpu.SemaphoreType.DMA,
            pltpu.SemaphoreType.DMA,
        ],
    )
    return fn(p, q, row, col)


_BE = 2000


def _emlp_body(g_ref, ef_ref, wf_ref, b1_ref, w2_ref, b2_ref, o_ref):
    pre = (g_ref[...]
           + jnp.dot(ef_ref[...].astype(jnp.bfloat16),
                     wf_ref[...].astype(jnp.bfloat16),
                     preferred_element_type=jnp.float32)
           + b1_ref[...])
    h = pre * jax.nn.sigmoid(pre)
    z = jnp.dot(h.astype(jnp.bfloat16), w2_ref[...].astype(jnp.bfloat16),
                preferred_element_type=jnp.float32) + b2_ref[...]
    o_ref[...] = z * jax.nn.sigmoid(z)


def _emlp(g, ef, wf, b1, w2, b2):
    return pl.pallas_call(
        _emlp_body,
        grid=(E // _BE,),
        in_specs=[
            pl.BlockSpec((_BE, D), lambda i: (i, 0)),
            pl.BlockSpec((_BE, F), lambda i: (i, 0)),
            pl.BlockSpec((F, D), lambda i: (0, 0)),
            pl.BlockSpec((1, D), lambda i: (0, 0)),
            pl.BlockSpec((D, D), lambda i: (0, 0)),
            pl.BlockSpec((1, D), lambda i: (0, 0)),
        ],
        out_specs=pl.BlockSpec((_BE, D), lambda i: (i, 0)),
        out_shape=jax.ShapeDtypeStruct((E, D), jnp.float32),
    )(g, ef, wf, b1, w2, b2)


def _scatter_body(m_hbm, row_hbm, zeros_hbm, out_hbm,
                  idxv, mbuf, acc, sem):
    c = lax.axis_index("c")
    s = lax.axis_index("s")
    pltpu.sync_copy(zeros_hbm, acc.at[pl.ds(s * NPT, NPT)])
    plsc.subcore_barrier()

    base = (c * NS + s) * EPW

    def chunk(k, carry):
        off = base + k * CH
        pltpu.sync_copy(row_hbm.at[pl.ds(off, CH)], idxv)
        pltpu.sync_copy(m_hbm.at[pl.ds(off, CH)], mbuf)
        pltpu.sync_copy(mbuf, acc.at[idxv], add=True)
        return carry

    lax.fori_loop(0, NCH, chunk, None)
    plsc.subcore_barrier()
    pltpu.sync_copy(acc.at[pl.ds(s * NPT, NPT)], out_hbm.at[c, pl.ds(s * NPT, NPT)])


def _scatter(m, row, zeros):
    fn = pl.kernel(
        _scatter_body,
        out_type=jax.ShapeDtypeStruct((NC, NPAD, D), jnp.float32),
        mesh=_mesh(),
        scratch_types=[
            pltpu.VMEM((CH,), jnp.int32),
            pltpu.VMEM((CH, D), jnp.float32),
            pltpu.VMEM_SHARED((NPAD, D), jnp.float32),
            pltpu.SemaphoreType.DMA,
        ],
    )
    return fn(m, row, zeros)


def _nmlp_body(x_ref, parts_ref, wx_ref, wa_ref, b1_ref, w2_ref, b2_ref, o_ref):
    agg = parts_ref[0] + parts_ref[1]
    pre = (jnp.dot(x_ref[...].astype(jnp.bfloat16),
                   wx_ref[...].astype(jnp.bfloat16),
                   preferred_element_type=jnp.float32)
           + jnp.dot(agg.astype(jnp.bfloat16), wa_ref[...].astype(jnp.bfloat16),
                     preferred_element_type=jnp.float32)
           + b1_ref[...])
    h = pre * jax.nn.sigmoid(pre)
    o_ref[...] = jnp.dot(h.astype(jnp.bfloat16), w2_ref[...].astype(jnp.bfloat16),
                         preferred_element_type=jnp.float32) + b2_ref[...]


def _nmlp(x, parts, wx, wa, b1, w2, b2):
    return pl.pallas_call(
        _nmlp_body,
        out_shape=jax.ShapeDtypeStruct((N, D), jnp.float32),
    )(x, parts, wx, wa, b1, w2, b2)


def kernel(x, edge_index, edge_feat, W1e, b1e, W2e, b2e, W1n, b1n, W2n, b2n):
    row = edge_index[0]
    col = edge_index[1]
    p, q = _pq(x, W1e[:D], W1e[D:2 * D])
    g = _gather(p, q, row, col)
    m = _emlp(g, edge_feat, W1e[2 * D:], b1e.reshape(1, D), W2e,
              b2e.reshape(1, D))
    parts = _scatter(m, row, jnp.zeros((NPT, D), jnp.float32))
    return _nmlp(x, parts[:, :N], W1n[:D], W1n[D:], b1n.reshape(1, D), W2n,
                 b2n.reshape(1, D))


# speedup vs baseline: 1.5269x; 1.5269x over previous
import functools

import numpy as np

import jax
import jax.numpy as jnp
from jax import lax
from jax.experimental import pallas as pl
from jax.experimental.pallas import tpu as pltpu
from jax.experimental.pallas import tpu_sc as plsc

N = 10000
E = 320000
D = 128
F = 16
LANES = 16
NC, NS = 2, 16
NW = NC * NS
NH = 2
EH = E // NH
EPW = EH // NW
CH = 40
NCH = EPW // CH
NPAD = 10240
NPT = NPAD // NS

_mesh = lambda: plsc.VectorSubcoreMesh(core_axis_name="c", subcore_axis_name="s")


def _pq_body(x_ref, wr_ref, wc_ref, p_ref, q_ref):
    xv = x_ref[...].astype(jnp.bfloat16)
    p_ref[...] = jnp.dot(xv, wr_ref[...].astype(jnp.bfloat16),
                         preferred_element_type=jnp.float32)
    q_ref[...] = jnp.dot(xv, wc_ref[...].astype(jnp.bfloat16),
                         preferred_element_type=jnp.float32)


def _pq(x, wr, wc):
    return pl.pallas_call(
        _pq_body,
        out_shape=[jax.ShapeDtypeStruct((N, D), jnp.float32),
                   jax.ShapeDtypeStruct((N, D), jnp.float32)],
    )(x, wr, wc)


def _gather_body(p_hbm, q_hbm, row_hbm, col_hbm, out_hbm,
                 idxr, idxc, buf_p, buf_q, buf_o, sems):
    wid = lax.axis_index("s") * NC + lax.axis_index("c")
    ebase = wid * EPW
    pltpu.sync_copy(row_hbm.at[wid], idxr)
    pltpu.sync_copy(col_hbm.at[wid], idxc)

    def issue(k, b):
        pltpu.async_copy(p_hbm.at[idxr.at[k]], buf_p.at[b], sems.at[b, 0])
        pltpu.async_copy(q_hbm.at[idxc.at[k]], buf_q.at[b], sems.at[b, 1])

    def drain(b):
        pltpu.make_async_copy(p_hbm.at[pl.ds(0, CH)], buf_p.at[b],
                              sems.at[b, 0]).wait()
        pltpu.make_async_copy(q_hbm.at[pl.ds(0, CH)], buf_q.at[b],
                              sems.at[b, 1]).wait()

    def add_store(k, b):
        def addrow(e, carry):
            for j in range(D // (2 * LANES)):
                lo = pl.ds(j * LANES, LANES)
                hi = pl.ds(D // 2 + j * LANES, LANES)
                va = buf_p[b, e, lo] + buf_q[b, e, lo]
                vb = buf_p[b, e, hi] + buf_q[b, e, hi]
                packed = plsc.pack(va, vb, format=plsc.PackFormat.INTERLEAVED)
                buf_o[b, e, lo] = plsc.bitcast(packed, jnp.int32)
            return carry

        lax.fori_loop(0, CH, addrow, None)
        pltpu.sync_copy(buf_o.at[b], out_hbm.at[pl.ds(ebase + k * CH, CH)])

    issue(0, 0)

    def pair(i, carry):
        k = i * 2
        drain(0)
        issue(k + 1, 1)
        add_store(k, 0)
        drain(1)
        issue(k + 2, 0)
        add_store(k + 1, 1)
        return carry

    lax.fori_loop(0, (NCH - 1) // 2, pair, None)
    drain(0)
    add_store(NCH - 1, 0)


def _gather(p, q, row3, col3):
    fn = pl.kernel(
        _gather_body,
        out_type=jax.ShapeDtypeStruct((EH, D // 2), jnp.int32),
        mesh=_mesh(),
        compiler_params=pltpu.CompilerParams(needs_layout_passes=False),
        scratch_types=[
            pltpu.VMEM((NCH, CH), jnp.int32),
            pltpu.VMEM((NCH, CH), jnp.int32),
            pltpu.VMEM((2, CH, D), jnp.float32),
            pltpu.VMEM((2, CH, D), jnp.float32),
            pltpu.VMEM((2, CH, D // 2), jnp.int32),
            pltpu.SemaphoreType.DMA((2, 2)),
        ],
    )
    return fn(p, q, row3, col3)


_BE = 1280


def _emlp_body(g_ref, eft_ref, wf_ref, b1_ref, w2_ref, b2_ref, o_ref):
    g32 = g_ref[...]
    gl = pltpu.bitcast(jnp.left_shift(g32, 16), jnp.float32)
    gh = pltpu.bitcast(jnp.bitwise_and(g32, jnp.int32(-65536)), jnp.float32)
    e1 = lax.dot_general(eft_ref[...].astype(jnp.bfloat16),
                         wf_ref[...].astype(jnp.bfloat16),
                         (((0,), (0,)), ((), ())),
                         preferred_element_type=jnp.float32)
    pre = jnp.concatenate([gl, gh], axis=1) + e1 + b1_ref[...]
    h = pre * jax.nn.sigmoid(pre)
    z = jnp.dot(h.astype(jnp.bfloat16), w2_ref[...].astype(jnp.bfloat16),
                preferred_element_type=jnp.float32) + b2_ref[...]
    o_ref[...] = z * jax.nn.sigmoid(z)


def _emlp(g, eft, wf, b1, w2, b2, half):
    return pl.pallas_call(
        _emlp_body,
        grid=(EH // _BE,),
        in_specs=[
            pl.BlockSpec((_BE, D // 2), lambda i: (i, 0)),
            pl.BlockSpec((F, _BE), lambda i, h=half: (0, i + h * (EH // _BE))),
            pl.BlockSpec((F, D), lambda i: (0, 0)),
            pl.BlockSpec((1, D), lambda i: (0, 0)),
            pl.BlockSpec((D, D), lambda i: (0, 0)),
            pl.BlockSpec((1, D), lambda i: (0, 0)),
        ],
        out_specs=pl.BlockSpec((_BE, D), lambda i: (i, 0)),
        out_shape=jax.ShapeDtypeStruct((EH, D), jnp.float32),
    )(g, eft, wf, b1, w2, b2)


def _scatter_body(m_hbm, row_hbm, zeros_hbm, out_hbm,
                  idxv, mbuf, acc, sems):
    c = lax.axis_index("c")
    s = lax.axis_index("s")
    wid = c * NS + s
    zcp = pltpu.async_copy(zeros_hbm, acc.at[pl.ds(s * NPT, NPT)], sems.at[2])
    pltpu.sync_copy(row_hbm.at[wid], idxv)
    zcp.wait()
    plsc.subcore_barrier()

    ebase = wid * EPW

    def issue(k, b):
        pltpu.async_copy(m_hbm.at[pl.ds(ebase + k * CH, CH)], mbuf.at[b],
                         sems.at[b])

    def drain(b):
        pltpu.make_async_copy(m_hbm.at[pl.ds(0, CH)], mbuf.at[b],
                              sems.at[b]).wait()

    def scat(k, b):
        pltpu.sync_copy(mbuf.at[b], acc.at[idxv.at[k]], add=True)

    issue(0, 0)

    def pair(i, carry):
        k = i * 2
        drain(0)
        issue(k + 1, 1)
        scat(k, 0)
        drain(1)
        issue(k + 2, 0)
        scat(k + 1, 1)
        return carry

    lax.fori_loop(0, (NCH - 1) // 2, pair, None)
    drain(0)
    scat(NCH - 1, 0)
    plsc.subcore_barrier()
    pltpu.sync_copy(acc.at[pl.ds(s * NPT, NPT)], out_hbm.at[c, pl.ds(s * NPT, NPT)])


def _scatter(m, row3, zeros):
    fn = pl.kernel(
        _scatter_body,
        out_type=jax.ShapeDtypeStruct((NC, NPAD, D), jnp.float32),
        mesh=_mesh(),
        scratch_types=[
            pltpu.VMEM((NCH, CH), jnp.int32),
            pltpu.VMEM((2, CH, D), jnp.float32),
            pltpu.VMEM_SHARED((NPAD, D), jnp.float32),
            pltpu.SemaphoreType.DMA((3,)),
        ],
    )
    return fn(m, row3, zeros)


def _nmlp_body(x_ref, pa_ref, pb_ref, wx_ref, wa_ref, b1_ref, w2_ref, b2_ref,
               o_ref):
    agg = (pa_ref[0, :N] + pa_ref[1, :N]) + (pb_ref[0, :N] + pb_ref[1, :N])
    pre = (jnp.dot(x_ref[...].astype(jnp.bfloat16),
                   wx_ref[...].astype(jnp.bfloat16),
                   preferred_element_type=jnp.float32)
           + jnp.dot(agg.astype(jnp.bfloat16), wa_ref[...].astype(jnp.bfloat16),
                     preferred_element_type=jnp.float32)
           + b1_ref[...])
    h = pre * jax.nn.sigmoid(pre)
    o_ref[...] = jnp.dot(h.astype(jnp.bfloat16), w2_ref[...].astype(jnp.bfloat16),
                         preferred_element_type=jnp.float32) + b2_ref[...]


def _nmlp(x, pa, pb, wx, wa, b1, w2, b2):
    return pl.pallas_call(
        _nmlp_body,
        out_shape=jax.ShapeDtypeStruct((N, D), jnp.float32),
    )(x, pa, pb, wx, wa, b1, w2, b2)


def kernel(x, edge_index, edge_feat, W1e, b1e, W2e, b2e, W1n, b1n, W2n, b2n):
    row4 = edge_index[0].reshape(NH, NW, NCH, CH)
    col4 = edge_index[1].reshape(NH, NW, NCH, CH)
    eft = edge_feat.T
    b1er = b1e.reshape(1, D)
    b2er = b2e.reshape(1, D)
    zeros = jnp.zeros((NPT, D), jnp.float32)
    p, q = _pq(x, W1e[:D], W1e[D:2 * D])
    g0 = _gather(p, q, row4[0], col4[0])
    g1 = _gather(p, q, row4[1], col4[1])
    m0 = _emlp(g0, eft, W1e[2 * D:], b1er, W2e, b2er, 0)
    m1 = _emlp(g1, eft, W1e[2 * D:], b1er, W2e, b2er, 1)
    pa = _scatter(m0, row4[0], zeros)
    pb = _scatter(m1, row4[1], zeros)
    return _nmlp(x, pa, pb, W1n[:D], W1n[D:], b1n.reshape(1, D), W2n,
                 b2n.reshape(1, D))


# speedup vs baseline: 1.8084x; 1.1844x over previous
import functools

import numpy as np

import jax
import jax.numpy as jnp
from jax import lax
from jax.experimental import pallas as pl
from jax.experimental.pallas import tpu as pltpu
from jax.experimental.pallas import tpu_sc as plsc

N = 10000
E = 320000
D = 128
F = 16
LANES = 16
NC, NS = 2, 16
NW = NC * NS
CH = 80
EPW0, EPW1 = 4800, 5200
E0, E1 = EPW0 * NW, EPW1 * NW
NPAD = 10240
NPT = NPAD // NS
_BE = 1280

_mesh = lambda: plsc.VectorSubcoreMesh(core_axis_name="c", subcore_axis_name="s")


def _pq_body(x_ref, wr_ref, wc_ref, p_ref, q_ref):
    xv = x_ref[...].astype(jnp.bfloat16)
    p_ref[...] = jnp.dot(xv, wr_ref[...].astype(jnp.bfloat16),
                         preferred_element_type=jnp.float32)
    q_ref[...] = jnp.dot(xv, wc_ref[...].astype(jnp.bfloat16),
                         preferred_element_type=jnp.float32)


def _pq(x, wr, wc):
    return pl.pallas_call(
        _pq_body,
        out_shape=[jax.ShapeDtypeStruct((N, D), jnp.float32),
                   jax.ShapeDtypeStruct((N, D), jnp.float32)],
    )(x, wr, wc)


def _make_gather(epw):
    nch = epw // CH
    eh = epw * NW

    def body(p_hbm, q_hbm, row_hbm, col_hbm, out_hbm,
             idxr, idxc, buf_p, buf_q, buf_o, sems):
        wid = lax.axis_index("s") * NC + lax.axis_index("c")
        ebase = wid * epw
        pltpu.sync_copy(row_hbm.at[wid], idxr)
        pltpu.sync_copy(col_hbm.at[wid], idxc)

        def issue(k, b):
            pltpu.async_copy(p_hbm.at[idxr.at[k]], buf_p.at[b], sems.at[b, 0])
            pltpu.async_copy(q_hbm.at[idxc.at[k]], buf_q.at[b], sems.at[b, 1])

        def drain(b):
            pltpu.make_async_copy(p_hbm.at[pl.ds(0, CH)], buf_p.at[b],
                                  sems.at[b, 0]).wait()
            pltpu.make_async_copy(q_hbm.at[pl.ds(0, CH)], buf_q.at[b],
                                  sems.at[b, 1]).wait()

        def add_store(k, b):
            def addrow(e, carry):
                for j in range(D // (2 * LANES)):
                    lo = pl.ds(j * LANES, LANES)
                    hi = pl.ds(D // 2 + j * LANES, LANES)
                    va = buf_p[b, e, lo] + buf_q[b, e, lo]
                    vb = buf_p[b, e, hi] + buf_q[b, e, hi]
                    pk = plsc.pack(va, vb, format=plsc.PackFormat.INTERLEAVED)
                    buf_o[b, e, lo] = plsc.bitcast(pk, jnp.int32)
                return carry

            lax.fori_loop(0, CH, addrow, None)
            pltpu.sync_copy(buf_o.at[b], out_hbm.at[pl.ds(ebase + k * CH, CH)])

        issue(0, 0)

        def pair(i, carry):
            k = i * 2
            drain(0)
            issue(k + 1, 1)
            add_store(k, 0)
            drain(1)
            issue(k + 2, 0)
            add_store(k + 1, 1)
            return carry

        if nch % 2:
            lax.fori_loop(0, (nch - 1) // 2, pair, None)
            drain(0)
            add_store(nch - 1, 0)
        else:
            lax.fori_loop(0, nch // 2 - 1, pair, None)
            k = nch - 2
            drain(0)
            issue(k + 1, 1)
            add_store(k, 0)
            drain(1)
            add_store(k + 1, 1)

    def call(p, q, row3, col3):
        fn = pl.kernel(
            body,
            out_type=jax.ShapeDtypeStruct((eh, D // 2), jnp.int32),
            mesh=_mesh(),
            compiler_params=pltpu.CompilerParams(needs_layout_passes=False),
            scratch_types=[
                pltpu.VMEM((nch, CH), jnp.int32),
                pltpu.VMEM((nch, CH), jnp.int32),
                pltpu.VMEM((2, CH, D), jnp.float32),
                pltpu.VMEM((2, CH, D), jnp.float32),
                pltpu.VMEM((2, CH, D // 2), jnp.int32),
                pltpu.SemaphoreType.DMA((2, 2)),
            ],
        )
        return fn(p, q, row3, col3)

    return call


_gather0 = _make_gather(EPW0)
_gather1 = _make_gather(EPW1)


def _emlp_body(g_ref, eft_ref, wf_ref, b1_ref, w2_ref, b2_ref, o_ref):
    g32 = g_ref[...]
    gl = pltpu.bitcast(jnp.left_shift(g32, 16), jnp.float32)
    gh = pltpu.bitcast(jnp.bitwise_and(g32, jnp.int32(-65536)), jnp.float32)
    e1 = lax.dot_general(eft_ref[...].astype(jnp.bfloat16),
                         wf_ref[...].astype(jnp.bfloat16),
                         (((0,), (0,)), ((), ())),
                         preferred_element_type=jnp.float32)
    pre = jnp.concatenate([gl, gh], axis=1) + e1 + b1_ref[...]
    h = pre * jax.nn.sigmoid(pre)
    z = jnp.dot(h.astype(jnp.bfloat16), w2_ref[...].astype(jnp.bfloat16),
                preferred_element_type=jnp.float32) + b2_ref[...]
    o_ref[...] = z * jax.nn.sigmoid(z)


def _emlp(g, eft, wf, b1, w2, b2, col_off, eh):
    return pl.pallas_call(
        _emlp_body,
        grid=(eh // _BE,),
        in_specs=[
            pl.BlockSpec((_BE, D // 2), lambda i: (i, 0)),
            pl.BlockSpec((F, _BE), lambda i: (0, i + col_off)),
            pl.BlockSpec((F, D), lambda i: (0, 0)),
            pl.BlockSpec((1, D), lambda i: (0, 0)),
            pl.BlockSpec((D, D), lambda i: (0, 0)),
            pl.BlockSpec((1, D), lambda i: (0, 0)),
        ],
        out_specs=pl.BlockSpec((_BE, D), lambda i: (i, 0)),
        out_shape=jax.ShapeDtypeStruct((eh, D), jnp.float32),
    )(g, eft, wf, b1, w2, b2)


def _make_scatter(epw):
    nch = epw // CH
    eh = epw * NW

    def body(m_hbm, row_hbm, zeros_hbm, out_hbm, idxv, mbuf, acc, sems):
        c = lax.axis_index("c")
        s = lax.axis_index("s")
        wid = c * NS + s
        zcp = pltpu.async_copy(zeros_hbm, acc.at[pl.ds(s * NPT, NPT)],
                               sems.at[2])
        pltpu.sync_copy(row_hbm.at[wid], idxv)
        zcp.wait()
        plsc.subcore_barrier()

        ebase = wid * epw

        def issue(k, b):
            pltpu.async_copy(m_hbm.at[pl.ds(ebase + k * CH, CH)], mbuf.at[b],
                             sems.at[b])

        def drain(b):
            pltpu.make_async_copy(m_hbm.at[pl.ds(0, CH)], mbuf.at[b],
                                  sems.at[b]).wait()

        def scat(k, b):
            pltpu.sync_copy(mbuf.at[b], acc.at[idxv.at[k]], add=True)

        issue(0, 0)

        def pair(i, carry):
            k = i * 2
            drain(0)
            issue(k + 1, 1)
            scat(k, 0)
            drain(1)
            issue(k + 2, 0)
            scat(k + 1, 1)
            return carry

        if nch % 2:
            lax.fori_loop(0, (nch - 1) // 2, pair, None)
            drain(0)
            scat(nch - 1, 0)
        else:
            lax.fori_loop(0, nch // 2 - 1, pair, None)
            k = nch - 2
            drain(0)
            issue(k + 1, 1)
            scat(k, 0)
            drain(1)
            scat(k + 1, 1)
        plsc.subcore_barrier()
        pltpu.sync_copy(acc.at[pl.ds(s * NPT, NPT)],
                        out_hbm.at[c, pl.ds(s * NPT, NPT)])

    def call(m, row3, zeros):
        fn = pl.kernel(
            body,
            out_type=jax.ShapeDtypeStruct((NC, NPAD, D), jnp.float32),
            mesh=_mesh(),
            scratch_types=[
                pltpu.VMEM((nch, CH), jnp.int32),
                pltpu.VMEM((2, CH, D), jnp.float32),
                pltpu.VMEM_SHARED((NPAD, D), jnp.float32),
                pltpu.SemaphoreType.DMA((3,)),
            ],
        )
        return fn(m, row3, zeros)

    return call


_scatter0 = _make_scatter(EPW0)
_scatter1 = _make_scatter(EPW1)


def _nmlp_body(x_ref, pa_ref, pb_ref, wx_ref, wa_ref, b1_ref, w2_ref, b2_ref,
               o_ref):
    agg = (pa_ref[0, :N] + pa_ref[1, :N]) + (pb_ref[0, :N] + pb_ref[1, :N])
    pre = (jnp.dot(x_ref[...].astype(jnp.bfloat16),
                   wx_ref[...].astype(jnp.bfloat16),
                   preferred_element_type=jnp.float32)
           + jnp.dot(agg.astype(jnp.bfloat16), wa_ref[...].astype(jnp.bfloat16),
                     preferred_element_type=jnp.float32)
           + b1_ref[...])
    h = pre * jax.nn.sigmoid(pre)
    o_ref[...] = jnp.dot(h.astype(jnp.bfloat16), w2_ref[...].astype(jnp.bfloat16),
                         preferred_element_type=jnp.float32) + b2_ref[...]


def _nmlp(x, pa, pb, wx, wa, b1, w2, b2):
    return pl.pallas_call(
        _nmlp_body,
        out_shape=jax.ShapeDtypeStruct((N, D), jnp.float32),
    )(x, pa, pb, wx, wa, b1, w2, b2)


def kernel(x, edge_index, edge_feat, W1e, b1e, W2e, b2e, W1n, b1n, W2n, b2n):
    row, col = edge_index[0], edge_index[1]
    r0 = row[:E0].reshape(NW, EPW0 // CH, CH)
    c0 = col[:E0].reshape(NW, EPW0 // CH, CH)
    r1 = row[E0:].reshape(NW, EPW1 // CH, CH)
    c1 = col[E0:].reshape(NW, EPW1 // CH, CH)
    eft = edge_feat.T
    wf = W1e[2 * D:]
    b1er = b1e.reshape(1, D)
    b2er = b2e.reshape(1, D)
    zeros = jnp.zeros((NPT, D), jnp.float32)
    p, q = _pq(x, W1e[:D], W1e[D:2 * D])
    g0 = _gather0(p, q, r0, c0)
    g1 = _gather1(p, q, r1, c1)
    m0 = _emlp(g0, eft, wf, b1er, W2e, b2er, 0, E0)
    m1 = _emlp(g1, eft, wf, b1er, W2e, b2er, E0 // _BE, E1)
    pa = _scatter0(m0, r0, zeros)
    pb = _scatter1(m1, r1, zeros)
    return _nmlp(x, pa, pb, W1n[:D], W1n[D:], b1n.reshape(1, D), W2n,
                 b2n.reshape(1, D))


# speedup vs baseline: 1.8091x; 1.0004x over previous
import functools

import numpy as np

import jax
import jax.numpy as jnp
from jax import lax
from jax.experimental import pallas as pl
from jax.experimental.pallas import tpu as pltpu
from jax.experimental.pallas import tpu_sc as plsc

N = 10000
E = 320000
D = 128
F = 16
LANES = 16
NC, NS = 2, 16
NW = NC * NS
CH = 80
EPW0, EPW1 = 5200, 4800
E0, E1 = EPW0 * NW, EPW1 * NW
NPT = 624
NLAST = N - 15 * NPT
_BE = 1280

_mesh = lambda: plsc.VectorSubcoreMesh(core_axis_name="c", subcore_axis_name="s")


def _pq_body(x_ref, wr_ref, wc_ref, p_ref, q_ref):
    xv = x_ref[...].astype(jnp.bfloat16)
    p_ref[...] = jnp.dot(xv, wr_ref[...].astype(jnp.bfloat16),
                         preferred_element_type=jnp.float32)
    q_ref[...] = jnp.dot(xv, wc_ref[...].astype(jnp.bfloat16),
                         preferred_element_type=jnp.float32)


def _pq(x, wr, wc):
    return pl.pallas_call(
        _pq_body,
        out_shape=[jax.ShapeDtypeStruct((N, D), jnp.float32),
                   jax.ShapeDtypeStruct((N, D), jnp.float32)],
    )(x, wr, wc)


def _make_gather(epw):
    nch = epw // CH
    eh = epw * NW

    def body(p_hbm, q_hbm, row_hbm, col_hbm, out_hbm,
             idxr, idxc, buf_p, buf_q, buf_o, sems):
        wid = lax.axis_index("s") * NC + lax.axis_index("c")
        ebase = wid * epw
        pltpu.sync_copy(row_hbm.at[wid], idxr)
        pltpu.sync_copy(col_hbm.at[wid], idxc)

        def issue(k, b):
            pltpu.async_copy(p_hbm.at[idxr.at[k]], buf_p.at[b], sems.at[b, 0])
            pltpu.async_copy(q_hbm.at[idxc.at[k]], buf_q.at[b], sems.at[b, 1])

        def drain(b):
            pltpu.make_async_copy(p_hbm.at[pl.ds(0, CH)], buf_p.at[b],
                                  sems.at[b, 0]).wait()
            pltpu.make_async_copy(q_hbm.at[pl.ds(0, CH)], buf_q.at[b],
                                  sems.at[b, 1]).wait()

        def add_store(k, b):
            def addrow(e, carry):
                for j in range(D // (2 * LANES)):
                    lo = pl.ds(j * LANES, LANES)
                    hi = pl.ds(D // 2 + j * LANES, LANES)
                    va = buf_p[b, e, lo] + buf_q[b, e, lo]
                    vb = buf_p[b, e, hi] + buf_q[b, e, hi]
                    pk = plsc.pack(va, vb, format=plsc.PackFormat.INTERLEAVED)
                    buf_o[b, e, lo] = plsc.bitcast(pk, jnp.int32)
                return carry

            lax.fori_loop(0, CH, addrow, None)
            pltpu.sync_copy(buf_o.at[b], out_hbm.at[pl.ds(ebase + k * CH, CH)])

        issue(0, 0)

        def pair(i, carry):
            k = i * 2
            drain(0)
            issue(k + 1, 1)
            add_store(k, 0)
            drain(1)
            issue(k + 2, 0)
            add_store(k + 1, 1)
            return carry

        if nch % 2:
            lax.fori_loop(0, (nch - 1) // 2, pair, None)
            drain(0)
            add_store(nch - 1, 0)
        else:
            lax.fori_loop(0, nch // 2 - 1, pair, None)
            k = nch - 2
            drain(0)
            issue(k + 1, 1)
            add_store(k, 0)
            drain(1)
            add_store(k + 1, 1)

    def call(p, q, row3, col3):
        fn = pl.kernel(
            body,
            out_type=jax.ShapeDtypeStruct((eh, D // 2), jnp.int32),
            mesh=_mesh(),
            compiler_params=pltpu.CompilerParams(needs_layout_passes=False),
            scratch_types=[
                pltpu.VMEM((nch, CH), jnp.int32),
                pltpu.VMEM((nch, CH), jnp.int32),
                pltpu.VMEM((2, CH, D), jnp.float32),
                pltpu.VMEM((2, CH, D), jnp.float32),
                pltpu.VMEM((2, CH, D // 2), jnp.int32),
                pltpu.SemaphoreType.DMA((2, 2)),
            ],
        )
        return fn(p, q, row3, col3)

    return call


_gather0 = _make_gather(EPW0)
_gather1 = _make_gather(EPW1)


def _emlp_body(g_ref, eft_ref, wf_ref, b1_ref, w2_ref, b2_ref, o_ref):
    g32 = g_ref[...]
    gl = pltpu.bitcast(jnp.left_shift(g32, 16), jnp.float32)
    gh = pltpu.bitcast(jnp.bitwise_and(g32, jnp.int32(-65536)), jnp.float32)
    e1 = lax.dot_general(eft_ref[...].astype(jnp.bfloat16),
                         wf_ref[...].astype(jnp.bfloat16),
                         (((0,), (0,)), ((), ())),
                         preferred_element_type=jnp.float32)
    pre = jnp.concatenate([gl, gh], axis=1) + e1 + b1_ref[...]
    h = pre * jax.nn.sigmoid(pre)
    z = jnp.dot(h.astype(jnp.bfloat16), w2_ref[...].astype(jnp.bfloat16),
                preferred_element_type=jnp.float32) + b2_ref[...]
    m = z * jax.nn.sigmoid(z)
    ui = pltpu.bitcast(m, jnp.int32)
    rnd = ui + 0x7FFF + jnp.bitwise_and(lax.shift_right_logical(ui, 16), 1)
    tl = lax.shift_right_logical(rnd[:, :D // 2], 16)
    th = jnp.bitwise_and(rnd[:, D // 2:], jnp.int32(-65536))
    o_ref[...] = jnp.bitwise_or(tl, th)


def _emlp(g, eft, wf, b1, w2, b2, col_off, eh):
    return pl.pallas_call(
        _emlp_body,
        grid=(eh // _BE,),
        in_specs=[
            pl.BlockSpec((_BE, D // 2), lambda i: (i, 0)),
            pl.BlockSpec((F, _BE), lambda i: (0, i + col_off)),
            pl.BlockSpec((F, D), lambda i: (0, 0)),
            pl.BlockSpec((1, D), lambda i: (0, 0)),
            pl.BlockSpec((D, D), lambda i: (0, 0)),
            pl.BlockSpec((1, D), lambda i: (0, 0)),
        ],
        out_specs=pl.BlockSpec((_BE, D // 2), lambda i: (i, 0)),
        out_shape=jax.ShapeDtypeStruct((eh, D // 2), jnp.int32),
    )(g, eft, wf, b1, w2, b2)


def _make_scatter(epw):
    nch = epw // CH
    eh = epw * NW

    def body(m_hbm, row_hbm, zeros_hbm, out_hbm, idxv, mbuf, fbuf, acc, sems):
        c = lax.axis_index("c")
        s = lax.axis_index("s")
        wid = c * NS + s

        @pl.when(s < NS - 1)
        def _():
            pltpu.sync_copy(zeros_hbm.at[pl.ds(0, NPT)],
                            acc.at[pl.ds(s * NPT, NPT)])

        @pl.when(s == NS - 1)
        def _():
            pltpu.sync_copy(zeros_hbm, acc.at[pl.ds((NS - 1) * NPT, NLAST)])

        pltpu.sync_copy(row_hbm.at[wid], idxv)
        plsc.subcore_barrier()

        ebase = wid * epw

        def issue_load(k, b):
            pltpu.async_copy(m_hbm.at[pl.ds(ebase + k * CH, CH)], mbuf.at[b],
                             sems.at[b])

        def drain_load(b):
            pltpu.make_async_copy(m_hbm.at[pl.ds(0, CH)], mbuf.at[b],
                                  sems.at[b]).wait()

        def unpack(b):
            def row(e, carry):
                for j in range(D // (2 * LANES)):
                    sl = pl.ds(j * LANES, LANES)
                    sh = pl.ds(D // 2 + j * LANES, LANES)
                    w = mbuf[b, e, sl]
                    fbuf[b, e, sl] = plsc.bitcast(
                        jnp.left_shift(w, 16), jnp.float32)
                    fbuf[b, e, sh] = plsc.bitcast(
                        jnp.bitwise_and(w, jnp.int32(-65536)), jnp.float32)
                return carry

            lax.fori_loop(0, CH, row, None)

        def issue_scat(k, b):
            pltpu.async_copy(fbuf.at[b], acc.at[idxv.at[k]], sems.at[2 + b],
                             add=True)

        def drain_scat(b):
            pltpu.make_async_copy(fbuf.at[b], acc.at[pl.ds(0, CH)],
                                  sems.at[2 + b]).wait()

        issue_load(0, 0)
        issue_load(1, 1)
        drain_load(0)
        unpack(0)
        issue_scat(0, 0)
        issue_load(2, 0)
        drain_load(1)
        unpack(1)
        issue_scat(1, 1)
        issue_load(3, 1)

        def pair(i, carry):
            k = i * 2

            def half(b):
                kk = k + b
                drain_load(b)
                drain_scat(b)
                unpack(b)
                issue_scat(kk, b)

                @pl.when(kk + 2 < nch)
                def _():
                    issue_load(kk + 2, b)

                return None

            half(0)
            half(1)
            return carry

        lax.fori_loop(1, nch // 2, pair, None)
        if nch % 2:
            drain_load(0)
            drain_scat(0)
            unpack(0)
            issue_scat(nch - 1, 0)
        drain_scat(0)
        drain_scat(1)
        plsc.subcore_barrier()

        @pl.when(s < NS - 1)
        def _():
            pltpu.sync_copy(acc.at[pl.ds(s * NPT, NPT)],
                            out_hbm.at[c, pl.ds(s * NPT, NPT)])

        @pl.when(s == NS - 1)
        def _():
            pltpu.sync_copy(acc.at[pl.ds((NS - 1) * NPT, NLAST)],
                            out_hbm.at[c, pl.ds((NS - 1) * NPT, NLAST)])

    def call(m, row3, zeros):
        fn = pl.kernel(
            body,
            out_type=jax.ShapeDtypeStruct((NC, N, D), jnp.float32),
            mesh=_mesh(),
            compiler_params=pltpu.CompilerParams(needs_layout_passes=False),
            scratch_types=[
                pltpu.VMEM((nch, CH), jnp.int32),
                pltpu.VMEM((2, CH, D // 2), jnp.int32),
                pltpu.VMEM((2, CH, D), jnp.float32),
                pltpu.VMEM_SHARED((N, D), jnp.float32),
                pltpu.SemaphoreType.DMA((4,)),
            ],
        )
        return fn(m, row3, zeros)

    return call


_scatter0 = _make_scatter(EPW0)
_scatter1 = _make_scatter(EPW1)


def _nmlp_body(x_ref, pa_ref, pb_ref, wx_ref, wa_ref, b1_ref, w2_ref, b2_ref,
               o_ref):
    agg = (pa_ref[0] + pa_ref[1]) + (pb_ref[0] + pb_ref[1])
    pre = (jnp.dot(x_ref[...].astype(jnp.bfloat16),
                   wx_ref[...].astype(jnp.bfloat16),
                   preferred_element_type=jnp.float32)
           + jnp.dot(agg.astype(jnp.bfloat16), wa_ref[...].astype(jnp.bfloat16),
                     preferred_element_type=jnp.float32)
           + b1_ref[...])
    h = pre * jax.nn.sigmoid(pre)
    o_ref[...] = jnp.dot(h.astype(jnp.bfloat16), w2_ref[...].astype(jnp.bfloat16),
                         preferred_element_type=jnp.float32) + b2_ref[...]


def _nmlp(x, pa, pb, wx, wa, b1, w2, b2):
    return pl.pallas_call(
        _nmlp_body,
        out_shape=jax.ShapeDtypeStruct((N, D), jnp.float32),
    )(x, pa, pb, wx, wa, b1, w2, b2)


def kernel(x, edge_index, edge_feat, W1e, b1e, W2e, b2e, W1n, b1n, W2n, b2n):
    row, col = edge_index[0], edge_index[1]
    r0 = row[:E0].reshape(NW, EPW0 // CH, CH)
    c0 = col[:E0].reshape(NW, EPW0 // CH, CH)
    r1 = row[E0:].reshape(NW, EPW1 // CH, CH)
    c1 = col[E0:].reshape(NW, EPW1 // CH, CH)
    eft = edge_feat.T
    wf = W1e[2 * D:]
    b1er = b1e.reshape(1, D)
    b2er = b2e.reshape(1, D)
    zeros = jnp.zeros((NLAST, D), jnp.float32)
    p, q = _pq(x, W1e[:D], W1e[D:2 * D])
    g0 = _gather0(p, q, r0, c0)
    g1 = _gather1(p, q, r1, c1)
    m0 = _emlp(g0, eft, wf, b1er, W2e, b2er, 0, E0)
    m1 = _emlp(g1, eft, wf, b1er, W2e, b2er, E0 // _BE, E1)
    pa = _scatter0(m0, r0, zeros)
    pb = _scatter1(m1, r1, zeros)
    return _nmlp(x, pa, pb, W1n[:D], W1n[D:], b1n.reshape(1, D), W2n,
                 b2n.reshape(1, D))


# speedup vs baseline: 1.8358x; 1.0147x over previous
import functools

import numpy as np

import jax
import jax.numpy as jnp
from jax import lax
from jax.experimental import pallas as pl
from jax.experimental.pallas import tpu as pltpu
from jax.experimental.pallas import tpu_sc as plsc

N = 10000
E = 320000
D = 128
F = 16
LANES = 16
NC, NS = 2, 16
NW = NC * NS
CH = 80
EPW0, EPW1, EPW2 = 4000, 4000, 2000
E0, E1, E2 = (EPW0 * NW, EPW1 * NW,
              EPW2 * NW)
NPT = 624
NLAST = N - 15 * NPT
_BE = 1280

_mesh = lambda: plsc.VectorSubcoreMesh(core_axis_name="c", subcore_axis_name="s")


def _pq_body(x_ref, wr_ref, wc_ref, p_ref, q_ref):
    xv = x_ref[...].astype(jnp.bfloat16)
    p_ref[...] = jnp.dot(xv, wr_ref[...].astype(jnp.bfloat16),
                         preferred_element_type=jnp.float32)
    q_ref[...] = jnp.dot(xv, wc_ref[...].astype(jnp.bfloat16),
                         preferred_element_type=jnp.float32)


def _pq(x, wr, wc):
    return pl.pallas_call(
        _pq_body,
        out_shape=[jax.ShapeDtypeStruct((N, D), jnp.float32),
                   jax.ShapeDtypeStruct((N, D), jnp.float32)],
    )(x, wr, wc)


def _make_gather(epw):
    nch = epw // CH
    eh = epw * NW

    def body(p_hbm, q_hbm, row_hbm, col_hbm, out_hbm,
             idxr, idxc, buf_p, buf_q, buf_o, sems):
        wid = lax.axis_index("s") * NC + lax.axis_index("c")
        ebase = wid * epw
        pltpu.sync_copy(row_hbm.at[wid], idxr)
        pltpu.sync_copy(col_hbm.at[wid], idxc)

        def issue(k, b):
            pltpu.async_copy(p_hbm.at[idxr.at[k]], buf_p.at[b], sems.at[b, 0])
            pltpu.async_copy(q_hbm.at[idxc.at[k]], buf_q.at[b], sems.at[b, 1])

        def drain(b):
            pltpu.make_async_copy(p_hbm.at[pl.ds(0, CH)], buf_p.at[b],
                                  sems.at[b, 0]).wait()
            pltpu.make_async_copy(q_hbm.at[pl.ds(0, CH)], buf_q.at[b],
                                  sems.at[b, 1]).wait()

        def add_store(k, b):
            def addrow(e, carry):
                for j in range(D // (2 * LANES)):
                    lo = pl.ds(j * LANES, LANES)
                    hi = pl.ds(D // 2 + j * LANES, LANES)
                    va = buf_p[b, e, lo] + buf_q[b, e, lo]
                    vb = buf_p[b, e, hi] + buf_q[b, e, hi]
                    pk = plsc.pack(va, vb, format=plsc.PackFormat.INTERLEAVED)
                    buf_o[b, e, lo] = plsc.bitcast(pk, jnp.int32)
                return carry

            lax.fori_loop(0, CH, addrow, None)
            pltpu.sync_copy(buf_o.at[b], out_hbm.at[pl.ds(ebase + k * CH, CH)])

        issue(0, 0)

        def pair(i, carry):
            k = i * 2
            drain(0)
            issue(k + 1, 1)
            add_store(k, 0)
            drain(1)
            issue(k + 2, 0)
            add_store(k + 1, 1)
            return carry

        if nch % 2:
            lax.fori_loop(0, (nch - 1) // 2, pair, None)
            drain(0)
            add_store(nch - 1, 0)
        else:
            lax.fori_loop(0, nch // 2 - 1, pair, None)
            k = nch - 2
            drain(0)
            issue(k + 1, 1)
            add_store(k, 0)
            drain(1)
            add_store(k + 1, 1)

    def call(p, q, row3, col3):
        fn = pl.kernel(
            body,
            out_type=jax.ShapeDtypeStruct((eh, D // 2), jnp.int32),
            mesh=_mesh(),
            compiler_params=pltpu.CompilerParams(needs_layout_passes=False),
            scratch_types=[
                pltpu.VMEM((nch, CH), jnp.int32),
                pltpu.VMEM((nch, CH), jnp.int32),
                pltpu.VMEM((2, CH, D), jnp.float32),
                pltpu.VMEM((2, CH, D), jnp.float32),
                pltpu.VMEM((2, CH, D // 2), jnp.int32),
                pltpu.SemaphoreType.DMA((2, 2)),
            ],
        )
        return fn(p, q, row3, col3)

    return call


_gather0 = _make_gather(EPW0)
_gather1 = _make_gather(EPW1)
_gather2 = _make_gather(EPW2)


def _emlp_body(g_ref, eft_ref, wf_ref, b1_ref, w2_ref, b2_ref, o_ref):
    g32 = g_ref[...]
    gl = pltpu.bitcast(jnp.left_shift(g32, 16), jnp.float32)
    gh = pltpu.bitcast(jnp.bitwise_and(g32, jnp.int32(-65536)), jnp.float32)
    e1 = lax.dot_general(eft_ref[...].astype(jnp.bfloat16),
                         wf_ref[...].astype(jnp.bfloat16),
                         (((0,), (0,)), ((), ())),
                         preferred_element_type=jnp.float32)
    pre = jnp.concatenate([gl, gh], axis=1) + e1 + b1_ref[...]
    h = pre * jax.nn.sigmoid(pre)
    z = jnp.dot(h.astype(jnp.bfloat16), w2_ref[...].astype(jnp.bfloat16),
                preferred_element_type=jnp.float32) + b2_ref[...]
    m = z * jax.nn.sigmoid(z)
    ui = pltpu.bitcast(m, jnp.int32)
    rnd = ui + 0x7FFF + jnp.bitwise_and(lax.shift_right_logical(ui, 16), 1)
    tl = lax.shift_right_logical(rnd[:, :D // 2], 16)
    th = jnp.bitwise_and(rnd[:, D // 2:], jnp.int32(-65536))
    o_ref[...] = jnp.bitwise_or(tl, th)


def _emlp(g, eft, wf, b1, w2, b2, col_off, eh):
    return pl.pallas_call(
        _emlp_body,
        grid=(eh // _BE,),
        in_specs=[
            pl.BlockSpec((_BE, D // 2), lambda i: (i, 0)),
            pl.BlockSpec((F, _BE), lambda i: (0, i + col_off)),
            pl.BlockSpec((F, D), lambda i: (0, 0)),
            pl.BlockSpec((1, D), lambda i: (0, 0)),
            pl.BlockSpec((D, D), lambda i: (0, 0)),
            pl.BlockSpec((1, D), lambda i: (0, 0)),
        ],
        out_specs=pl.BlockSpec((_BE, D // 2), lambda i: (i, 0)),
        out_shape=jax.ShapeDtypeStruct((eh, D // 2), jnp.int32),
    )(g, eft, wf, b1, w2, b2)


def _make_scatter(epw):
    nch = epw // CH
    eh = epw * NW

    def body(m_hbm, row_hbm, zeros_hbm, out_hbm, idxv, mbuf, fbuf, acc, sems):
        c = lax.axis_index("c")
        s = lax.axis_index("s")
        wid = c * NS + s

        @pl.when(s < NS - 1)
        def _():
            pltpu.sync_copy(zeros_hbm.at[pl.ds(0, NPT)],
                            acc.at[pl.ds(s * NPT, NPT)])

        @pl.when(s == NS - 1)
        def _():
            pltpu.sync_copy(zeros_hbm, acc.at[pl.ds((NS - 1) * NPT, NLAST)])

        pltpu.sync_copy(row_hbm.at[wid], idxv)
        plsc.subcore_barrier()

        ebase = wid * epw

        def issue_load(k, b):
            pltpu.async_copy(m_hbm.at[pl.ds(ebase + k * CH, CH)], mbuf.at[b],
                             sems.at[b])

        def drain_load(b):
            pltpu.make_async_copy(m_hbm.at[pl.ds(0, CH)], mbuf.at[b],
                                  sems.at[b]).wait()

        def unpack(b):
            def row(e, carry):
                for j in range(D // (2 * LANES)):
                    sl = pl.ds(j * LANES, LANES)
                    sh = pl.ds(D // 2 + j * LANES, LANES)
                    w = mbuf[b, e, sl]
                    fbuf[b, e, sl] = plsc.bitcast(
                        jnp.left_shift(w, 16), jnp.float32)
                    fbuf[b, e, sh] = plsc.bitcast(
                        jnp.bitwise_and(w, jnp.int32(-65536)), jnp.float32)
                return carry

            lax.fori_loop(0, CH, row, None)

        def issue_scat(k, b):
            pltpu.async_copy(fbuf.at[b], acc.at[idxv.at[k]], sems.at[2 + b],
                             add=True)

        def drain_scat(b):
            pltpu.make_async_copy(fbuf.at[b], acc.at[pl.ds(0, CH)],
                                  sems.at[2 + b]).wait()

        issue_load(0, 0)
        issue_load(1, 1)
        drain_load(0)
        unpack(0)
        issue_scat(0, 0)
        issue_load(2, 0)
        drain_load(1)
        unpack(1)
        issue_scat(1, 1)
        issue_load(3, 1)

        def pair(i, carry):
            k = i * 2

            def half(b):
                kk = k + b
                drain_load(b)
                drain_scat(b)
                unpack(b)
                issue_scat(kk, b)

                @pl.when(kk + 2 < nch)
                def _():
                    issue_load(kk + 2, b)

                return None

            half(0)
            half(1)
            return carry

        lax.fori_loop(1, nch // 2, pair, None)
        if nch % 2:
            drain_load(0)
            drain_scat(0)
            unpack(0)
            issue_scat(nch - 1, 0)
        drain_scat(0)
        drain_scat(1)
        plsc.subcore_barrier()

        @pl.when(s < NS - 1)
        def _():
            pltpu.sync_copy(acc.at[pl.ds(s * NPT, NPT)],
                            out_hbm.at[c, pl.ds(s * NPT, NPT)])

        @pl.when(s == NS - 1)
        def _():
            pltpu.sync_copy(acc.at[pl.ds((NS - 1) * NPT, NLAST)],
                            out_hbm.at[c, pl.ds((NS - 1) * NPT, NLAST)])

    def call(m, row3, zeros):
        fn = pl.kernel(
            body,
            out_type=jax.ShapeDtypeStruct((NC, N, D), jnp.float32),
            mesh=_mesh(),
            compiler_params=pltpu.CompilerParams(needs_layout_passes=False),
            scratch_types=[
                pltpu.VMEM((nch, CH), jnp.int32),
                pltpu.VMEM((2, CH, D // 2), jnp.int32),
                pltpu.VMEM((2, CH, D), jnp.float32),
                pltpu.VMEM_SHARED((N, D), jnp.float32),
                pltpu.SemaphoreType.DMA((4,)),
            ],
        )
        return fn(m, row3, zeros)

    return call


_scatter0 = _make_scatter(EPW0)
_scatter1 = _make_scatter(EPW1)
_scatter2 = _make_scatter(EPW2)


def _nmlp_body(x_ref, pa_ref, pb_ref, pc_ref, wx_ref, wa_ref, b1_ref, w2_ref,
               b2_ref, o_ref):
    agg = ((pa_ref[0] + pa_ref[1]) + (pb_ref[0] + pb_ref[1])
           + (pc_ref[0] + pc_ref[1]))
    pre = (jnp.dot(x_ref[...].astype(jnp.bfloat16),
                   wx_ref[...].astype(jnp.bfloat16),
                   preferred_element_type=jnp.float32)
           + jnp.dot(agg.astype(jnp.bfloat16), wa_ref[...].astype(jnp.bfloat16),
                     preferred_element_type=jnp.float32)
           + b1_ref[...])
    h = pre * jax.nn.sigmoid(pre)
    o_ref[...] = jnp.dot(h.astype(jnp.bfloat16), w2_ref[...].astype(jnp.bfloat16),
                         preferred_element_type=jnp.float32) + b2_ref[...]


def _nmlp(x, pa, pb, pc, wx, wa, b1, w2, b2):
    return pl.pallas_call(
        _nmlp_body,
        out_shape=jax.ShapeDtypeStruct((N, D), jnp.float32),
    )(x, pa, pb, pc, wx, wa, b1, w2, b2)


def kernel(x, edge_index, edge_feat, W1e, b1e, W2e, b2e, W1n, b1n, W2n, b2n):
    row, col = edge_index[0], edge_index[1]
    r0 = row[:E0].reshape(NW, EPW0 // CH, CH)
    c0 = col[:E0].reshape(NW, EPW0 // CH, CH)
    r1 = row[E0:E0 + E1].reshape(NW, EPW1 // CH, CH)
    c1 = col[E0:E0 + E1].reshape(NW, EPW1 // CH, CH)
    r2 = row[E0 + E1:].reshape(NW, EPW2 // CH, CH)
    c2 = col[E0 + E1:].reshape(NW, EPW2 // CH, CH)
    eft = edge_feat.T
    wf = W1e[2 * D:]
    b1er = b1e.reshape(1, D)
    b2er = b2e.reshape(1, D)
    zeros = jnp.zeros((NLAST, D), jnp.float32)
    p, q = _pq(x, W1e[:D], W1e[D:2 * D])
    g0 = _gather0(p, q, r0, c0)
    g1 = _gather1(p, q, r1, c1)
    g2 = _gather2(p, q, r2, c2)
    m0 = _emlp(g0, eft, wf, b1er, W2e, b2er, 0, E0)
    m1 = _emlp(g1, eft, wf, b1er, W2e, b2er, E0 // _BE, E1)
    m2 = _emlp(g2, eft, wf, b1er, W2e, b2er, (E0 + E1) // _BE, E2)
    pa = _scatter0(m0, r0, zeros)
    pb = _scatter1(m1, r1, zeros)
    pc = _scatter2(m2, r2, zeros)
    return _nmlp(x, pa, pb, pc, W1n[:D], W1n[D:], b1n.reshape(1, D), W2n,
                 b2n.reshape(1, D))
